# baseline (speedup 1.0000x reference)
import jax
import jax.numpy as jnp
from jax import lax
from jax.experimental import pallas as pl
from jax.experimental.pallas import tpu as pltpu
from jax.experimental.pallas import tpu_sc as plsc

_K = 65536
_DIM = 128
_B = 1024

_NC = 2
_NS = 16
_NW = _NC * _NS
_SLAB = _K // _NW
_CROWS = 128
_NB = 6
_PRIME = 4


def _pipe_copy(pairs, bufs, in_sems, out_sems):
    n = len(pairs)
    prime = min(_PRIME, n)
    ins = [None] * n
    outs = [None] * n

    def fire_in(j):
        return pltpu.async_copy(pairs[j][0], bufs[j % _NB],
                                in_sems.at[j % _NB])

    for j in range(prime):
        ins[j] = fire_in(j)
    drained = set()
    for j in range(n):
        nxt = j + prime
        if nxt < n:
            prev = nxt - _NB
            if prev >= 0:
                outs[prev].wait()
                drained.add(prev)
            ins[nxt] = fire_in(nxt)
        ins[j].wait()
        outs[j] = pltpu.async_copy(bufs[j % _NB], pairs[j][1],
                                   out_sems.at[j % _NB])
    for j in range(n):
        if j not in drained:
            outs[j].wait()


def _chunks(src, dst, rows, src_base, dst_base):
    return [(src.at[pl.ds(src_base + r, _CROWS), :],
             dst.at[pl.ds(dst_base + r, _CROWS), :])
            for r in range(0, rows, _CROWS)]


def _sc_body(ptr_hbm, keys_hbm, q_hbm, out_hbm, p_v, bufs, in_sems,
             out_sems):
    sid = lax.axis_index("s")
    wid = sid * _NC + lax.axis_index("c")
    base = wid * _SLAB
    bufs = [b.at[sid] for b in bufs]

    pltpu.sync_copy(ptr_hbm, p_v)
    p = pl.multiple_of(p_v[...][0], _B)

    _pipe_copy(_chunks(q_hbm, out_hbm, _SLAB, base, base),
               bufs, in_sems, out_sems)

    @pl.when(jnp.logical_and(p >= base, p < base + _SLAB))
    def _():
        _pipe_copy(_chunks(keys_hbm, out_hbm, _B, 0, p),
                   bufs, in_sems, out_sems)


def _sc_enqueue(keys, queue, ptr):
    mesh = plsc.VectorSubcoreMesh(core_axis_name="c", subcore_axis_name="s")
    run = pl.kernel(
        _sc_body,
        out_type=jax.ShapeDtypeStruct((_K, _DIM), jnp.float32),
        mesh=mesh,
        scratch_types=[
            pltpu.VMEM((16,), jnp.int32),
            [pltpu.VMEM_SHARED((_NS, _CROWS, _DIM), jnp.float32)
             for _ in range(_NB)],
            pltpu.SemaphoreType.DMA((_NB,)),
            pltpu.SemaphoreType.DMA((_NB,)),
        ],
    )
    ptr_v = jnp.broadcast_to(ptr, (16,)).astype(jnp.int32)
    return run(ptr_v, keys, queue)


_TROWS = 4096
_TCH = _K // _TROWS


def _tc_body(ptr_smem, q_hbm, keys_hbm, out_hbm, sem, ksem):
    copies = [
        pltpu.make_async_copy(
            q_hbm.at[pl.ds(c * _TROWS, _TROWS), :],
            out_hbm.at[pl.ds(c * _TROWS, _TROWS), :], sem)
        for c in range(_TCH)
    ]
    for c in copies:
        c.start()
    for c in copies:
        c.wait()
    p = pl.multiple_of(ptr_smem[0], _B)
    kcopy = pltpu.make_async_copy(keys_hbm, out_hbm.at[pl.ds(p, _B), :], ksem)
    kcopy.start()
    kcopy.wait()


def _tc_enqueue(keys, queue, ptr):
    return pl.pallas_call(
        _tc_body,
        in_specs=[
            pl.BlockSpec(memory_space=pltpu.SMEM),
            pl.BlockSpec(memory_space=pl.ANY),
            pl.BlockSpec(memory_space=pl.ANY),
        ],
        out_specs=pl.BlockSpec(memory_space=pl.ANY),
        out_shape=jax.ShapeDtypeStruct((_K, _DIM), jnp.float32),
        scratch_shapes=[pltpu.SemaphoreType.DMA, pltpu.SemaphoreType.DMA],
    )(ptr.astype(jnp.int32), queue, keys)


def kernel(keys_1, keys_2, queue_1, queue_2, queue_1_ptr, queue_2_ptr):
    q2_new = _sc_enqueue(keys_2, queue_2, queue_2_ptr)
    q1_new = _tc_enqueue(keys_1, queue_1, queue_1_ptr)
    ptr1_new = ((queue_1_ptr + _B) % _K).astype(jnp.int32)
    ptr2_new = ((queue_2_ptr + _B) % _K).astype(jnp.int32)
    return q1_new, q2_new, ptr1_new, ptr2_new


# speedup vs baseline: 9.3515x; 9.3515x over previous
import jax
import jax.numpy as jnp
from jax import lax
from jax.experimental import pallas as pl
from jax.experimental.pallas import tpu as pltpu
from jax.experimental.pallas import tpu_sc as plsc

_K = 65536
_DIM = 128
_B = 1024

_NC = 2
_NS = 16
_NW = _NC * _NS
_SLAB = _K // _NW
_CROWS = 128
_NB = 6
_PRIME = 4


def _pipe_copy(pairs, bufs, in_sems, out_sems):
    n = len(pairs)
    prime = min(_PRIME, n)
    ins = [None] * n
    outs = [None] * n

    def fire_in(j):
        return pltpu.async_copy(pairs[j][0], bufs[j % _NB],
                                in_sems.at[j % _NB])

    for j in range(prime):
        ins[j] = fire_in(j)
    drained = set()
    for j in range(n):
        nxt = j + prime
        if nxt < n:
            prev = nxt - _NB
            if prev >= 0:
                outs[prev].wait()
                drained.add(prev)
            ins[nxt] = fire_in(nxt)
        ins[j].wait()
        outs[j] = pltpu.async_copy(bufs[j % _NB], pairs[j][1],
                                   out_sems.at[j % _NB])
    for j in range(n):
        if j not in drained:
            outs[j].wait()


def _chunks(src, dst, rows, src_base, dst_base):
    return [(src.at[pl.ds(src_base + r, _CROWS), :],
             dst.at[pl.ds(dst_base + r, _CROWS), :])
            for r in range(0, rows, _CROWS)]


def _sc_body(ptr_hbm, keys_hbm, q_hbm, out_hbm, p_v, bufs, in_sems,
             out_sems, p_sem):
    sid = lax.axis_index("s")
    wid = sid * _NC + lax.axis_index("c")
    base = wid * _SLAB
    bufs = [b.at[sid] for b in bufs]

    pcopy = pltpu.async_copy(ptr_hbm, p_v, p_sem)
    _pipe_copy(_chunks(q_hbm, out_hbm, _SLAB, base, base),
               bufs, in_sems, out_sems)
    pcopy.wait()
    p = pl.multiple_of(p_v[...][0], _B)

    @pl.when(jnp.logical_and(p >= base, p < base + _SLAB))
    def _():
        _pipe_copy(_chunks(keys_hbm, out_hbm, _B, 0, p),
                   bufs, in_sems, out_sems)


def _sc_enqueue(keys, queue, ptr):
    mesh = plsc.VectorSubcoreMesh(core_axis_name="c", subcore_axis_name="s")
    run = pl.kernel(
        _sc_body,
        out_type=jax.ShapeDtypeStruct((_K, _DIM), jnp.float32),
        mesh=mesh,
        scratch_types=[
            pltpu.VMEM((16,), jnp.int32),
            [pltpu.VMEM_SHARED((_NS, _CROWS, _DIM), jnp.float32)
             for _ in range(_NB)],
            pltpu.SemaphoreType.DMA((_NB,)),
            pltpu.SemaphoreType.DMA((_NB,)),
            pltpu.SemaphoreType.DMA,
        ],
    )
    ptr_v = jnp.broadcast_to(ptr, (16,)).astype(jnp.int32)
    return run(ptr_v, keys, queue)


_TROWS = 512


def _tc_body(ptr_smem, q_ref, keys_ref, out_ref):
    i = pl.program_id(0)
    wblk = ptr_smem[0] // _TROWS

    @pl.when(i == wblk)
    def _():
        out_ref[...] = keys_ref[pl.ds(0, _TROWS), :]

    @pl.when(i == wblk + 1)
    def _():
        out_ref[...] = keys_ref[pl.ds(_B - _TROWS, _TROWS), :]

    @pl.when(jnp.logical_or(i < wblk, i > wblk + 1))
    def _():
        out_ref[...] = q_ref[...]


def _tc_enqueue(keys, queue, ptr):
    return pl.pallas_call(
        _tc_body,
        grid=_K // _TROWS,
        in_specs=[
            pl.BlockSpec(memory_space=pltpu.SMEM),
            pl.BlockSpec((_TROWS, _DIM), lambda i: (i, 0)),
            pl.BlockSpec((_B, _DIM), lambda i: (0, 0)),
        ],
        out_specs=pl.BlockSpec((_TROWS, _DIM), lambda i: (i, 0)),
        out_shape=jax.ShapeDtypeStruct((_K, _DIM), jnp.float32),
    )(ptr.astype(jnp.int32), queue, keys)


def kernel(keys_1, keys_2, queue_1, queue_2, queue_1_ptr, queue_2_ptr):
    q2_new = _sc_enqueue(keys_2, queue_2, queue_2_ptr)
    q1_new = _tc_enqueue(keys_1, queue_1, queue_1_ptr)
    ptr1_new = ((queue_1_ptr + _B) % _K).astype(jnp.int32)
    ptr2_new = ((queue_2_ptr + _B) % _K).astype(jnp.int32)
    return q1_new, q2_new, ptr1_new, ptr2_new


# speedup vs baseline: 22.2890x; 2.3835x over previous
import jax
import jax.numpy as jnp
from jax.experimental import pallas as pl
from jax.experimental.pallas import tpu as pltpu

_K = 65536
_DIM = 128
_B = 1024

_TROWS = 4096
_TNB = 8
_TPRIME = 6


def _tc_pipe(pairs, rows, bufs, in_sems, out_sems):
    n = len(pairs)
    prime = min(_TPRIME, n)
    ins = [None] * n
    outs = [None] * n

    def fire_in(j):
        b = j % _TNB
        return pltpu.async_copy(pairs[j][0], bufs[b].at[pl.ds(0, rows), :],
                                in_sems.at[b])

    for j in range(prime):
        ins[j] = fire_in(j)
    drained = set()
    for j in range(n):
        b = j % _TNB
        nxt = j + prime
        if nxt < n:
            prev = nxt - _TNB
            if prev >= 0:
                outs[prev].wait()
                drained.add(prev)
            ins[nxt] = fire_in(nxt)
        ins[j].wait()
        outs[j] = pltpu.async_copy(bufs[b].at[pl.ds(0, rows), :],
                                   pairs[j][1], out_sems.at[b])
    for j in range(n):
        if j not in drained:
            outs[j].wait()


def _tc_body(p1_smem, p2_smem, q1_hbm, keys1_hbm, q2_hbm, keys2_hbm,
             out1_hbm, out2_hbm, bufs, kbufs, in_sems, out_sems, ksems):
    kins = [pltpu.async_copy(keys1_hbm, kbufs[0], ksems.at[0]),
            pltpu.async_copy(keys2_hbm, kbufs[1], ksems.at[1])]

    bulk = []
    for c in range(_K // _TROWS):
        sl = pl.ds(c * _TROWS, _TROWS)
        bulk.append((q1_hbm.at[sl, :], out1_hbm.at[sl, :]))
        bulk.append((q2_hbm.at[sl, :], out2_hbm.at[sl, :]))
    _tc_pipe(bulk, _TROWS, bufs, in_sems, out_sems)

    p1 = pl.multiple_of(p1_smem[0], _B)
    p2 = pl.multiple_of(p2_smem[0], _B)
    kins[0].wait()
    kins[1].wait()
    kouts = [
        pltpu.async_copy(kbufs[0], out1_hbm.at[pl.ds(p1, _B), :],
                         ksems.at[0]),
        pltpu.async_copy(kbufs[1], out2_hbm.at[pl.ds(p2, _B), :],
                         ksems.at[1]),
    ]
    kouts[0].wait()
    kouts[1].wait()


def _enqueue2(keys_1, queue_1, ptr_1, keys_2, queue_2, ptr_2):
    return pl.pallas_call(
        _tc_body,
        in_specs=[
            pl.BlockSpec(memory_space=pltpu.SMEM),
            pl.BlockSpec(memory_space=pltpu.SMEM),
            pl.BlockSpec(memory_space=pl.ANY),
            pl.BlockSpec(memory_space=pl.ANY),
            pl.BlockSpec(memory_space=pl.ANY),
            pl.BlockSpec(memory_space=pl.ANY),
        ],
        out_specs=(pl.BlockSpec(memory_space=pl.ANY),
                   pl.BlockSpec(memory_space=pl.ANY)),
        out_shape=(jax.ShapeDtypeStruct((_K, _DIM), jnp.float32),
                   jax.ShapeDtypeStruct((_K, _DIM), jnp.float32)),
        scratch_shapes=[
            [pltpu.VMEM((_TROWS, _DIM), jnp.float32) for _ in range(_TNB)],
            [pltpu.VMEM((_B, _DIM), jnp.float32) for _ in range(2)],
            pltpu.SemaphoreType.DMA((_TNB,)),
            pltpu.SemaphoreType.DMA((_TNB,)),
            pltpu.SemaphoreType.DMA((2,)),
        ],
    )(ptr_1.astype(jnp.int32), ptr_2.astype(jnp.int32),
      queue_1, keys_1, queue_2, keys_2)


def kernel(keys_1, keys_2, queue_1, queue_2, queue_1_ptr, queue_2_ptr):
    q1_new, q2_new = _enqueue2(keys_1, queue_1, queue_1_ptr,
                               keys_2, queue_2, queue_2_ptr)
    ptr1_new = ((queue_1_ptr + _B) % _K).astype(jnp.int32)
    ptr2_new = ((queue_2_ptr + _B) % _K).astype(jnp.int32)
    return q1_new, q2_new, ptr1_new, ptr2_new


# speedup vs baseline: 22.2994x; 1.0005x over previous
import jax
import jax.numpy as jnp
from jax.experimental import pallas as pl
from jax.experimental.pallas import tpu as pltpu

_K = 65536
_DIM = 128
_B = 1024

_TROWS = 4096
_TNB = 12
_TPRIME = 8


def _tc_pipe(pairs, rows, bufs, in_sems, out_sems):
    n = len(pairs)
    prime = min(_TPRIME, n)
    ins = [None] * n
    outs = [None] * n

    def fire_in(j):
        b = j % _TNB
        return pltpu.async_copy(pairs[j][0], bufs[b].at[pl.ds(0, rows), :],
                                in_sems.at[b])

    for j in range(prime):
        ins[j] = fire_in(j)
    drained = set()
    for j in range(n):
        b = j % _TNB
        nxt = j + prime
        if nxt < n:
            prev = nxt - _TNB
            if prev >= 0:
                outs[prev].wait()
                drained.add(prev)
            ins[nxt] = fire_in(nxt)
        ins[j].wait()
        outs[j] = pltpu.async_copy(bufs[b].at[pl.ds(0, rows), :],
                                   pairs[j][1], out_sems.at[b])
    for j in range(n):
        if j not in drained:
            outs[j].wait()


def _tc_body(p1_smem, p2_smem, q1_hbm, keys1_hbm, q2_hbm, keys2_hbm,
             out1_hbm, out2_hbm, bufs, kbufs, in_sems, out_sems, ksems):
    kins = [pltpu.async_copy(keys1_hbm, kbufs[0], ksems.at[0]),
            pltpu.async_copy(keys2_hbm, kbufs[1], ksems.at[1])]

    bulk = []
    for c in range(_K // _TROWS):
        sl = pl.ds(c * _TROWS, _TROWS)
        bulk.append((q1_hbm.at[sl, :], out1_hbm.at[sl, :]))
        bulk.append((q2_hbm.at[sl, :], out2_hbm.at[sl, :]))
    _tc_pipe(bulk, _TROWS, bufs, in_sems, out_sems)

    p1 = pl.multiple_of(p1_smem[0], _B)
    p2 = pl.multiple_of(p2_smem[0], _B)
    kins[0].wait()
    kins[1].wait()
    kouts = [
        pltpu.async_copy(kbufs[0], out1_hbm.at[pl.ds(p1, _B), :],
                         ksems.at[0]),
        pltpu.async_copy(kbufs[1], out2_hbm.at[pl.ds(p2, _B), :],
                         ksems.at[1]),
    ]
    kouts[0].wait()
    kouts[1].wait()


def _enqueue2(keys_1, queue_1, ptr_1, keys_2, queue_2, ptr_2):
    return pl.pallas_call(
        _tc_body,
        in_specs=[
            pl.BlockSpec(memory_space=pltpu.SMEM),
            pl.BlockSpec(memory_space=pltpu.SMEM),
            pl.BlockSpec(memory_space=pl.ANY),
            pl.BlockSpec(memory_space=pl.ANY),
            pl.BlockSpec(memory_space=pl.ANY),
            pl.BlockSpec(memory_space=pl.ANY),
        ],
        out_specs=(pl.BlockSpec(memory_space=pl.ANY),
                   pl.BlockSpec(memory_space=pl.ANY)),
        out_shape=(jax.ShapeDtypeStruct((_K, _DIM), jnp.float32),
                   jax.ShapeDtypeStruct((_K, _DIM), jnp.float32)),
        scratch_shapes=[
            [pltpu.VMEM((_TROWS, _DIM), jnp.float32) for _ in range(_TNB)],
            [pltpu.VMEM((_B, _DIM), jnp.float32) for _ in range(2)],
            pltpu.SemaphoreType.DMA((_TNB,)),
            pltpu.SemaphoreType.DMA((_TNB,)),
            pltpu.SemaphoreType.DMA((2,)),
        ],
    )(ptr_1.astype(jnp.int32), ptr_2.astype(jnp.int32),
      queue_1, keys_1, queue_2, keys_2)


def kernel(keys_1, keys_2, queue_1, queue_2, queue_1_ptr, queue_2_ptr):
    q1_new, q2_new = _enqueue2(keys_1, queue_1, queue_1_ptr,
                               keys_2, queue_2, queue_2_ptr)
    ptr1_new = ((queue_1_ptr + _B) % _K).astype(jnp.int32)
    ptr2_new = ((queue_2_ptr + _B) % _K).astype(jnp.int32)
    return q1_new, q2_new, ptr1_new, ptr2_new
